# Initial kernel scaffold; baseline (speedup 1.0000x reference)
import jax
import jax.numpy as jnp
from jax.experimental import pallas as pl


def kernel(x, edge_index, gamma, beta, W1, att_src1, att_dst1, b1, W2, att_src2, att_dst2, b2):
    raise NotImplementedError("write your pallas kernel here")


# speedup vs baseline: 1.1696x; 1.1696x over previous
import functools
import jax
import jax.numpy as jnp
from jax.experimental import pallas as pl
from jax.experimental.pallas import tpu as pltpu

N = 10000
E = 320000
D = 128
H1 = 8
O1 = 64


def _tc1_body(x_ref, g_ref, b_ref, w_ref, as_ref, ad_ref, h_ref, al_ref):
    x = x_ref[...]
    mu = jnp.mean(x, axis=-1, keepdims=True)
    var = jnp.mean((x - mu) ** 2, axis=-1, keepdims=True)
    xn = (x - mu) * jax.lax.rsqrt(var + 1e-5) * g_ref[...] + b_ref[...]
    h = jnp.dot(xn, w_ref[...], preferred_element_type=jnp.float32)
    h_ref[...] = h
    al_s = jnp.dot(h, as_ref[...], preferred_element_type=jnp.float32)
    al_d = jnp.dot(h, ad_ref[...], preferred_element_type=jnp.float32)
    al_ref[...] = jnp.concatenate([al_s, al_d], axis=-1)


def _dense1(x, gamma, beta, W1, att_src1, att_dst1):
    A_s = (jnp.eye(H1, dtype=jnp.float32)[:, None, :] * att_src1[:, :, None]).reshape(
        H1 * O1, H1)
    A_d = (jnp.eye(H1, dtype=jnp.float32)[:, None, :] * att_dst1[:, :, None]).reshape(
        H1 * O1, H1)
    blk = 1000
    grid = (N // blk,)
    return pl.pallas_call(
        _tc1_body,
        grid=grid,
        in_specs=[
            pl.BlockSpec((blk, D), lambda i: (i, 0)),
            pl.BlockSpec((1, D), lambda i: (0, 0)),
            pl.BlockSpec((1, D), lambda i: (0, 0)),
            pl.BlockSpec((D, H1 * O1), lambda i: (0, 0)),
            pl.BlockSpec((H1 * O1, H1), lambda i: (0, 0)),
            pl.BlockSpec((H1 * O1, H1), lambda i: (0, 0)),
        ],
        out_specs=[
            pl.BlockSpec((blk, H1 * O1), lambda i: (i, 0)),
            pl.BlockSpec((blk, 2 * H1), lambda i: (i, 0)),
        ],
        out_shape=[
            jax.ShapeDtypeStruct((N, H1 * O1), jnp.float32),
            jax.ShapeDtypeStruct((N, 2 * H1), jnp.float32),
        ],
    )(x, gamma.reshape(1, D), beta.reshape(1, D), W1, A_s, A_d)


def _gat_edges(h, al_s, al_d, src, dst, heads):
    e = jax.nn.leaky_relu(al_s[src] + al_d[dst], 0.2)
    ex = jnp.exp(e)
    s = jax.ops.segment_sum(ex, dst, num_segments=N)
    hh = h.reshape(N, heads, -1)
    raw = jax.ops.segment_sum(ex[:, :, None] * hh[src], dst, num_segments=N)
    e_self = jax.nn.leaky_relu(al_s + al_d, 0.2)
    ex_self = jnp.exp(e_self)
    s = s + ex_self
    raw = raw + ex_self[:, :, None] * hh
    return raw / (s[:, :, None] + 1e-16)


def kernel(x, edge_index, gamma, beta, W1, att_src1, att_dst1, b1, W2,
           att_src2, att_dst2, b2):
    src = edge_index[0]
    dst = edge_index[1]
    H, al1 = _dense1(x, gamma, beta, W1, att_src1, att_dst1)
    out1 = _gat_edges(H, al1[:, :H1], al1[:, H1:], src, dst, H1)
    h2 = jax.nn.elu(out1.reshape(N, H1 * O1) + b1)
    H2 = h2 @ W2
    al2_s = H2 @ att_src2[0]
    al2_d = H2 @ att_dst2[0]
    out2 = _gat_edges(H2, al2_s[:, None], al2_d[:, None], src, dst, 1)
    return out2.reshape(N, D) + b2
